# Initial kernel scaffold; baseline (speedup 1.0000x reference)
import jax
import jax.numpy as jnp
from jax.experimental import pallas as pl


def kernel(hidden_states, router_weight, w1, w2, w3):
    raise NotImplementedError("write your pallas kernel here")


# speedup vs baseline: 1.1334x; 1.1334x over previous
import functools
import jax
import jax.numpy as jnp
from jax.experimental import pallas as pl
from jax.experimental.pallas import tpu as pltpu

INTERPRET = False

F32 = jnp.float32
BF16 = jnp.bfloat16


def _router_body(x_ref, rw_ref, wn_ref, *, E, K):
    x = x_ref[...]
    rw = rw_ref[...]
    logits = jax.lax.dot_general(x, rw, (((1,), (1,)), ((), ())),
                                 preferred_element_type=F32)
    m = jnp.max(logits, axis=1, keepdims=True)
    ex = jnp.exp(logits - m)
    probs = ex / jnp.sum(ex, axis=1, keepdims=True)

    T = probs.shape[0]
    iota = jax.lax.broadcasted_iota(jnp.int32, (T, E), 1)
    masked = probs
    selected = jnp.zeros((T, E), dtype=jnp.bool_)
    for _ in range(K):
        mx = jnp.max(masked, axis=1, keepdims=True)
        is_max = masked == mx
        cand = jnp.where(is_max, iota, E)
        first = jnp.min(cand, axis=1, keepdims=True)
        newly = iota == first
        selected = jnp.logical_or(selected, newly)
        masked = jnp.where(newly, -jnp.inf, masked)

    psel = jnp.where(selected, probs, 0.0)
    wsum = jnp.sum(psel, axis=1, keepdims=True)
    wn_ref[...] = psel / wsum


def _layer1_body(x_ref, w1_ref, h1_ref):
    xb = x_ref[...].astype(BF16)
    wb = w1_ref[0].astype(BF16)
    h = jax.lax.dot_general(xb, wb, (((1,), (1,)), ((), ())),
                            preferred_element_type=F32)
    h1_ref[0] = (h * jax.nn.sigmoid(h)).astype(BF16)


def _layer2_body(h1_ref, w2_ref, h2_ref):
    hb = h1_ref[0]
    wb = w2_ref[0].astype(BF16)
    h = jax.lax.dot_general(hb, wb, (((1,), (1,)), ((), ())),
                            preferred_element_type=F32)
    h2_ref[0] = (h * jax.nn.sigmoid(h)).astype(BF16)


def _layer3_body(h2_ref, w3_ref, wn_ref, out_ref, acc_ref, *, E, TB):
    e = pl.program_id(0)
    t = pl.program_id(1)
    hb = h2_ref[0]
    wb = w3_ref[0].astype(BF16)
    y = jax.lax.dot_general(hb, wb, (((1,), (1,)), ((), ())),
                            preferred_element_type=F32)
    wn = wn_ref[...]
    iota = jax.lax.broadcasted_iota(jnp.int32, wn.shape, 1)
    scale = jnp.sum(jnp.where(iota == e, wn, 0.0), axis=1)
    contrib = scale[:, None] * y
    sl = pl.ds(t * TB, TB)

    @pl.when(e == 0)
    def _():
        acc_ref[sl, :] = contrib

    @pl.when(e > 0)
    def _():
        acc_ref[sl, :] += contrib

    @pl.when(e == E - 1)
    def _():
        out_ref[...] = acc_ref[sl, :]


def kernel(hidden_states, router_weight, w1, w2, w3):
    B, S, H = hidden_states.shape
    E, F, _ = w1.shape
    T = B * S
    K = 4
    TB = 256
    NTB = T // TB

    x = hidden_states.reshape(T, H)

    w_norm = pl.pallas_call(
        functools.partial(_router_body, E=E, K=K),
        out_shape=jax.ShapeDtypeStruct((T, E), F32),
        interpret=INTERPRET,
    )(x, router_weight)

    h1 = pl.pallas_call(
        _layer1_body,
        grid=(E, NTB),
        in_specs=[
            pl.BlockSpec((TB, H), lambda e, t: (t, 0)),
            pl.BlockSpec((1, F, H), lambda e, t: (e, 0, 0)),
        ],
        out_specs=pl.BlockSpec((1, TB, F), lambda e, t: (e, t, 0)),
        out_shape=jax.ShapeDtypeStruct((E, T, F), BF16),
        interpret=INTERPRET,
    )(x, w1)

    h2 = pl.pallas_call(
        _layer2_body,
        grid=(E, NTB),
        in_specs=[
            pl.BlockSpec((1, TB, F), lambda e, t: (e, t, 0)),
            pl.BlockSpec((1, F, F), lambda e, t: (e, 0, 0)),
        ],
        out_specs=pl.BlockSpec((1, TB, F), lambda e, t: (e, t, 0)),
        out_shape=jax.ShapeDtypeStruct((E, T, F), BF16),
        interpret=INTERPRET,
    )(h1, w2)

    out = pl.pallas_call(
        functools.partial(_layer3_body, E=E, TB=TB),
        grid=(E, NTB),
        in_specs=[
            pl.BlockSpec((1, TB, F), lambda e, t: (e, t, 0)),
            pl.BlockSpec((1, H, F), lambda e, t: (e, 0, 0)),
            pl.BlockSpec((TB, E), lambda e, t: (t, 0)),
        ],
        out_specs=pl.BlockSpec((TB, H), lambda e, t: (t, 0)),
        out_shape=jax.ShapeDtypeStruct((T, H), F32),
        scratch_shapes=[pltpu.VMEM((T, H), F32)],
        interpret=INTERPRET,
    )(h2, w3, w_norm)

    return out.reshape(B, S, H)


# speedup vs baseline: 1.6176x; 1.4272x over previous
import functools
import jax
import jax.numpy as jnp
from jax.experimental import pallas as pl
from jax.experimental.pallas import tpu as pltpu
from jax.experimental.pallas import tpu_sc as plsc

F32 = jnp.float32
BF16 = jnp.bfloat16
I32 = jnp.int32

K = 4
BLK = 256
RW = 128


def _router_body(x_ref, rw_ref, ws_ref, ds_ref, meta_ref, *, E, T, NB):
    x = x_ref[...]
    rw = rw_ref[...]
    logits = jax.lax.dot_general(x, rw, (((1,), (1,)), ((), ())),
                                 preferred_element_type=F32)
    m = jnp.max(logits, axis=1, keepdims=True)
    ex = jnp.exp(logits - m)
    probs = ex / jnp.sum(ex, axis=1, keepdims=True)

    iota = jax.lax.broadcasted_iota(I32, (T, E), 1)
    masked = probs
    selected = jnp.zeros((T, E), dtype=jnp.bool_)
    firsts = []
    for _ in range(K):
        mx = jnp.max(masked, axis=1, keepdims=True)
        is_max = masked == mx
        cand = jnp.where(is_max, iota, E)
        first = jnp.min(cand, axis=1, keepdims=True)
        firsts.append(first)
        newly = iota == first
        selected = jnp.logical_or(selected, newly)
        masked = jnp.where(newly, -jnp.inf, masked)

    self32 = selected.astype(F32)
    psel = jnp.where(selected, probs, 0.0)
    wsum = jnp.sum(psel, axis=1, keepdims=True)
    wnorm = psel / wsum

    ranks_parts = []
    CH = 256
    for i in range(T // CH):
        row = jax.lax.broadcasted_iota(I32, (CH, T), 0) + i * CH
        col = jax.lax.broadcasted_iota(I32, (CH, T), 1)
        ltri = (col < row).astype(F32)
        ranks_parts.append(jax.lax.dot_general(
            ltri, self32, (((1,), (0,)), ((), ())),
            preferred_element_type=F32))
    ranks = jnp.concatenate(ranks_parts, axis=0)

    counts = jnp.sum(self32, axis=0, keepdims=True)
    cnt = counts.astype(I32)
    pc = ((cnt + (BLK - 1)) // BLK) * BLK
    r16 = jax.lax.broadcasted_iota(I32, (E, E), 0)
    c16 = jax.lax.broadcasted_iota(I32, (E, E), 1)
    l16 = (r16 < c16).astype(F32)
    offs = jax.lax.dot_general(pc.astype(F32), l16, (((1,), (0,)), ((), ())),
                               preferred_element_type=F32).astype(I32)

    dest = offs + ranks.astype(I32)

    col8 = jax.lax.broadcasted_iota(I32, (T, 2 * K), 1)
    ws = jnp.zeros((T, 2 * K), F32)
    ds = jnp.zeros((T, 2 * K), I32)
    for r in range(K):
        first = firsts[r]
        w_r = jnp.sum(jnp.where(iota == first, wnorm, 0.0), axis=1,
                      keepdims=True)
        d_r = jnp.sum(jnp.where(iota == first, dest, 0), axis=1,
                      keepdims=True)
        ws = jnp.where(col8 == r, w_r, ws)
        ds = jnp.where(col8 == r, d_r, ds)
    ws_ref[...] = ws
    ds_ref[...] = ds

    tp = offs[0, E - 1] + pc[0, E - 1]
    bcol = jax.lax.broadcasted_iota(I32, (8, 64), 1)
    row_start = bcol * BLK
    def expert_of(rows):
        acc = jnp.zeros((8, 64), I32)
        for e in range(E):
            inr = jnp.logical_and(rows >= offs[0, e], rows < offs[0, e] + pc[0, e])
            acc = acc + jnp.where(inr, e, 0)
        return acc

    be_raw = expert_of(row_start)
    be_last = expert_of(jnp.full((8, 64), tp - BLK, I32))
    active = row_start < tp
    be_fill = jnp.where(active, be_raw, be_last)
    rowi = jax.lax.broadcasted_iota(I32, (8, 64), 0)
    meta = jnp.where(rowi == 0, be_fill, 0)
    meta = jnp.where(rowi == 1, active.astype(I32), meta)
    meta_ref[...] = meta


def _make_router(E, T, NB):
    return pl.pallas_call(
        functools.partial(_router_body, E=E, T=T, NB=NB),
        out_shape=[
            jax.ShapeDtypeStruct((T, 2 * K), F32),
            jax.ShapeDtypeStruct((T, 2 * K), I32),
            jax.ShapeDtypeStruct((8, 64), I32),
        ],
    )


def _scatter_rows(x2, dest2, P2, HW):
    T2 = x2.shape[0]
    mesh = plsc.VectorSubcoreMesh(core_axis_name="core",
                                  subcore_axis_name="subcore")
    nch = T2 // RW

    @functools.partial(
        pl.kernel,
        out_type=jax.ShapeDtypeStruct((P2, HW), F32),
        mesh=mesh,
    )
    def scat(x_hbm, i_hbm, o_hbm):
        def body(x_vmem, i_vmem):
            pltpu.sync_copy(x_vmem, o_hbm.at[i_vmem.at[0]])

        pltpu.emit_pipeline(
            body,
            grid=(K, nch),
            in_specs=[
                pl.BlockSpec((RW, HW), lambda k, c: (c, 0)),
                pl.BlockSpec((1, RW), lambda k, c: (0, k * nch + c)),
            ],
            out_specs=[],
            core_axis_name=("core", "subcore"),
            dimension_semantics=(pltpu.PARALLEL, pltpu.PARALLEL),
        )(x_hbm, i_hbm)

    return scat(x2, dest2)


def _gather_rows(y2, dest2, HW):
    A = dest2.shape[1]
    mesh = plsc.VectorSubcoreMesh(core_axis_name="core",
                                  subcore_axis_name="subcore")

    @functools.partial(
        pl.kernel,
        out_type=jax.ShapeDtypeStruct((A, HW), F32),
        mesh=mesh,
    )
    def gath(y_hbm, i_hbm, o_hbm):
        def body(i_vmem, o_vmem):
            pltpu.sync_copy(y_hbm.at[i_vmem.at[0]], o_vmem)

        pltpu.emit_pipeline(
            body,
            grid=(A // RW,),
            in_specs=[pl.BlockSpec((1, RW), lambda i: (0, i))],
            out_specs=[pl.BlockSpec((RW, HW), lambda i: (i, 0))],
            core_axis_name=("core", "subcore"),
            dimension_semantics=(pltpu.PARALLEL,),
        )(i_hbm, o_hbm)

    return gath(y2, dest2)


def _l1_body(be_ref, act_ref, x_ref, w_ref, o_ref):
    b = pl.program_id(0)

    @pl.when(act_ref[b] == 1)
    def _():
        xb = x_ref[...].astype(BF16)
        wb = w_ref[0].astype(BF16)
        h = jax.lax.dot_general(xb, wb, (((1,), (1,)), ((), ())),
                                preferred_element_type=F32)
        o_ref[...] = (h * jax.nn.sigmoid(h)).astype(BF16)


def _l2_body(be_ref, act_ref, h_ref, w_ref, o_ref):
    b = pl.program_id(0)

    @pl.when(act_ref[b] == 1)
    def _():
        hb = h_ref[...]
        wb = w_ref[0].astype(BF16)
        h = jax.lax.dot_general(hb, wb, (((1,), (1,)), ((), ())),
                                preferred_element_type=F32)
        o_ref[...] = (h * jax.nn.sigmoid(h)).astype(BF16)


def _l3_body(be_ref, act_ref, h_ref, w_ref, o_ref):
    b = pl.program_id(0)

    @pl.when(act_ref[b] == 1)
    def _():
        hb = h_ref[...]
        wb = w_ref[0].astype(BF16)
        y = jax.lax.dot_general(hb, wb, (((1,), (1,)), ((), ())),
                                preferred_element_type=F32)
        o_ref[...] = y


def _layer_pass(body, xin, w, be, act, NB, out_cols, out_dtype):
    P = NB * BLK
    Wd = w.shape[1]
    return pl.pallas_call(
        body,
        grid_spec=pltpu.PrefetchScalarGridSpec(
            num_scalar_prefetch=2,
            grid=(NB,),
            in_specs=[
                pl.BlockSpec((BLK, xin.shape[1]), lambda b, be, act: (b, 0)),
                pl.BlockSpec((1, Wd, w.shape[2]),
                             lambda b, be, act: (be[b], 0, 0)),
            ],
            out_specs=pl.BlockSpec((BLK, out_cols), lambda b, be, act: (b, 0)),
        ),
        out_shape=jax.ShapeDtypeStruct((P, out_cols), out_dtype),
    )(be, act, xin, w)


def _combine_body(y_ref, w_ref, o_ref):
    y4 = y_ref[...]
    wn = w_ref[...]
    acc = wn[:, 0:1] * y4[0]
    for k in range(1, K):
        acc = acc + wn[:, k:k + 1] * y4[k]
    o_ref[...] = acc


def kernel(hidden_states, router_weight, w1, w2, w3):
    B, S, H = hidden_states.shape
    E, F, _ = w1.shape
    T = B * S
    P = T * K + E * BLK
    NB = P // BLK
    TB = 256

    x = hidden_states.reshape(T, H)

    w_slot, dest_slot, meta = _make_router(E, T, NB)(x, router_weight)

    be = meta[0, :NB]
    act = meta[1, :NB]
    dest_flat = dest_slot[:, :K].T
    HW = H // 2
    dest2 = jnp.stack([2 * dest_flat, 2 * dest_flat + 1],
                      axis=-1).reshape(1, 2 * K * T)

    x_sorted = _scatter_rows(x.reshape(2 * T, HW), dest2, 2 * P, HW)

    h1 = _layer_pass(_l1_body, x_sorted.reshape(P, H), w1, be, act, NB, F, BF16)
    h2 = _layer_pass(_l2_body, h1, w2, be, act, NB, F, BF16)
    y_sorted = _layer_pass(_l3_body, h2, w3, be, act, NB, H, F32)

    ygat = _gather_rows(y_sorted.reshape(2 * P, HW), dest2, HW).reshape(K, T, H)

    out = pl.pallas_call(
        _combine_body,
        grid=(T // TB,),
        in_specs=[
            pl.BlockSpec((K, TB, H), lambda t: (0, t, 0)),
            pl.BlockSpec((TB, 2 * K), lambda t: (t, 0)),
        ],
        out_specs=pl.BlockSpec((TB, H), lambda t: (t, 0)),
        out_shape=jax.ShapeDtypeStruct((T, H), F32),
    )(ygat, w_slot)

    return out.reshape(B, S, H)
